# baseline (speedup 1.0000x reference)
import jax
import jax.numpy as jnp
from jax.experimental import pallas as pl
from jax.experimental.pallas import tpu as pltpu


def _gcn_fused_kernel(
    x_ref, w1_ref, b1_ref, w2_ref, b2_ref, p_hbm_ref,
    out_ref, pv_ref, a_ref, sem0, sem1,
):
    n = pv_ref.shape[0]
    m = n // 2
    c0 = pltpu.make_async_copy(
        p_hbm_ref.at[pl.ds(0, m), :], pv_ref.at[pl.ds(0, m), :], sem0
    )
    c1 = pltpu.make_async_copy(
        p_hbm_ref.at[pl.ds(m, m), :], pv_ref.at[pl.ds(m, m), :], sem1
    )
    c0.start()
    c1.start()
    xw = jnp.dot(x_ref[...], w1_ref[...], preferred_element_type=jnp.float32)

    row_i = jax.lax.broadcasted_iota(jnp.int32, (m, m), 0)
    col_i = jax.lax.broadcasted_iota(jnp.int32, (m, m), 1)
    eye = jnp.where(row_i == col_i, jnp.float32(1.0), jnp.float32(0.0))

    c0.wait()
    p00 = pv_ref[0:m, 0:m]
    a_ref[0:m, 0:m] = jax.nn.sigmoid(eye + 0.5 * (p00 + p00.T))

    c1.wait()
    a01 = jax.nn.sigmoid(0.5 * (pv_ref[0:m, m:n] + pv_ref[m:n, 0:m].T))
    a_ref[0:m, m:n] = a01
    a_ref[m:n, 0:m] = a01.T
    p11 = pv_ref[m:n, m:n]
    a_ref[m:n, m:n] = jax.nn.sigmoid(eye + 0.5 * (p11 + p11.T))

    a = a_ref[...]
    dis = jax.lax.rsqrt(jnp.sum(a, axis=1, keepdims=True))
    h = jnp.maximum(
        dis * jnp.dot(a, dis * xw, preferred_element_type=jnp.float32)
        + b1_ref[...],
        0.0,
    )
    hw = dis * jnp.dot(h, w2_ref[...], preferred_element_type=jnp.float32)
    out_ref[...] = (
        dis * jnp.dot(a, hw, preferred_element_type=jnp.float32) + b2_ref[...]
    )


@jax.jit
def kernel(x, adaptive_params, W1, b1, W2, b2):
    n, din = x.shape
    hid = W1.shape[1]
    dout = W2.shape[1]
    return pl.pallas_call(
        _gcn_fused_kernel,
        in_specs=[
            pl.BlockSpec((n, din), lambda: (0, 0)),
            pl.BlockSpec((din, hid), lambda: (0, 0)),
            pl.BlockSpec((1, hid), lambda: (0, 0)),
            pl.BlockSpec((hid, dout), lambda: (0, 0)),
            pl.BlockSpec((1, dout), lambda: (0, 0)),
            pl.BlockSpec(memory_space=pl.ANY),
        ],
        out_specs=pl.BlockSpec((n, dout), lambda: (0, 0)),
        scratch_shapes=[
            pltpu.VMEM((n, n), jnp.float32),
            pltpu.VMEM((n, n), jnp.float32),
            pltpu.SemaphoreType.DMA,
            pltpu.SemaphoreType.DMA,
        ],
        out_shape=jax.ShapeDtypeStruct((n, dout), x.dtype),
    )(x, W1, b1.reshape(1, -1), W2, b2.reshape(1, -1), adaptive_params)


# speedup vs baseline: 1.0566x; 1.0566x over previous
import jax
import jax.numpy as jnp
from jax.experimental import pallas as pl


def _gcn_fused_kernel(x_ref, p_ref, w1_ref, b1_ref, w2_ref, b2_ref, out_ref):
    p = p_ref[...]
    n = p.shape[0]
    row_i = jax.lax.broadcasted_iota(jnp.int32, (n, n), 0)
    col_i = jax.lax.broadcasted_iota(jnp.int32, (n, n), 1)
    eye = jnp.where(row_i == col_i, jnp.float32(1.0), jnp.float32(0.0))
    a = jax.nn.sigmoid(eye + 0.5 * (p + p.T))
    dis = jax.lax.rsqrt(jnp.sum(a, axis=1, keepdims=True))
    xw = jnp.dot(x_ref[...], w1_ref[...], preferred_element_type=jnp.float32)
    h = jnp.maximum(
        dis * jnp.dot(a, dis * xw, preferred_element_type=jnp.float32)
        + b1_ref[...],
        0.0,
    )
    hw = dis * jnp.dot(h, w2_ref[...], preferred_element_type=jnp.float32)
    out_ref[...] = (
        dis * jnp.dot(a, hw, preferred_element_type=jnp.float32) + b2_ref[...]
    )


@jax.jit
def kernel(x, adaptive_params, W1, b1, W2, b2):
    n = x.shape[0]
    return pl.pallas_call(
        _gcn_fused_kernel,
        out_shape=jax.ShapeDtypeStruct((n, W2.shape[1]), x.dtype),
    )(x, adaptive_params, W1, b1.reshape(1, -1), W2, b2.reshape(1, -1))
